# baseline (speedup 1.0000x reference)
import functools

import jax
import jax.numpy as jnp
from jax import lax
from jax.experimental import pallas as pl
from jax.experimental.pallas import tpu as pltpu
from jax.experimental.pallas import tpu_sc as plsc

_B = 16
_C = 3
_H = 384
_W = 384
_NC = 2
_NS = 16
_NW = _NC * _NS
_HALF = _H // 2
_RC = 32
_NCH = _HALF // _RC
_VPR = _W // 16
_EPS2 = 1e-16


def _sc_body(o_hbm, t_hbm, acc_out, cnt_out,
             bo0, bo1, bo2, bt0, bt1, bt2, stage):
    cid = lax.axis_index("c")
    sid = lax.axis_index("s")
    wid = sid * _NC + cid
    b = wid // 2
    half = wid % 2

    def chunk(ci, carry):
        r0 = half * _HALF + ci * _RC
        pltpu.sync_copy(o_hbm.at[b, 0, pl.ds(r0, _RC), :], bo0)
        pltpu.sync_copy(o_hbm.at[b, 1, pl.ds(r0, _RC), :], bo1)
        pltpu.sync_copy(o_hbm.at[b, 2, pl.ds(r0, _RC), :], bo2)
        pltpu.sync_copy(t_hbm.at[b, 0, pl.ds(r0, _RC), :], bt0)
        pltpu.sync_copy(t_hbm.at[b, 1, pl.ds(r0, _RC), :], bt1)
        pltpu.sync_copy(t_hbm.at[b, 2, pl.ds(r0, _RC), :], bt2)

        def row(i, rcarry):
            def vec(j, vcarry):
                acc, cnt = vcarry
                sl = pl.ds(j * 16, 16)
                o0 = bo0[i, sl]
                o1 = bo1[i, sl]
                o2 = bo2[i, sl]
                t0 = bt0[i, sl]
                t1 = bt1[i, sl]
                t2 = bt2[i, sl]
                dot = o0 * t0 + o1 * t1 + o2 * t2
                no2 = o0 * o0 + o1 * o1 + o2 * o2
                nt2 = t0 * t0 + t1 * t1 + t2 * t2
                p = jnp.maximum(no2, _EPS2) * jnp.maximum(nt2, _EPS2)
                iv = lax.bitcast_convert_type(p, jnp.int32)
                iv = 0x5F3759DF - lax.shift_right_arithmetic(iv, 1)
                y = lax.bitcast_convert_type(iv, jnp.float32)
                ph = 0.5 * p
                y = y * (1.5 - ph * y * y)
                y = y * (1.5 - ph * y * y)
                y = y * (1.5 - ph * y * y)
                m = nt2 > 0.0
                acc = acc + jnp.where(m, 1.0 - dot * y, 0.0)
                cnt = cnt + jnp.where(m, 1.0, 0.0)
                return acc, cnt
            return lax.fori_loop(0, _VPR, vec, rcarry)
        return lax.fori_loop(0, _RC, row, carry)

    zero = jnp.zeros((16,), jnp.float32)
    acc, cnt = lax.fori_loop(0, _NCH, chunk, (zero, zero))
    stage[...] = acc
    pltpu.sync_copy(stage, acc_out.at[wid])
    stage[...] = cnt
    pltpu.sync_copy(stage, cnt_out.at[wid])


def _fin_body(a_ref, c_ref, out_ref):
    loss = jnp.sum(a_ref[...]) / jnp.sum(c_ref[...])
    out_ref[...] = loss.reshape(1, 1)


def kernel(output, target):
    mesh = plsc.VectorSubcoreMesh(core_axis_name="c", subcore_axis_name="s")
    sc = functools.partial(
        pl.kernel,
        mesh=mesh,
        out_type=[
            jax.ShapeDtypeStruct((_NW, 16), jnp.float32),
            jax.ShapeDtypeStruct((_NW, 16), jnp.float32),
        ],
        scratch_types=[pltpu.VMEM((_RC, _W), jnp.float32)] * 6
        + [pltpu.VMEM((16,), jnp.float32)],
    )(_sc_body)
    acc_p, cnt_p = sc(output, target)
    out = pl.pallas_call(
        _fin_body,
        out_shape=jax.ShapeDtypeStruct((1, 1), jnp.float32),
    )(acc_p, cnt_p)
    return out[0, 0]


# speedup vs baseline: 1.5175x; 1.5175x over previous
import functools

import jax
import jax.numpy as jnp
from jax import lax
from jax.experimental import pallas as pl
from jax.experimental.pallas import tpu as pltpu
from jax.experimental.pallas import tpu_sc as plsc

_B = 16
_C = 3
_H = 384
_W = 384
_NC = 2
_NS = 16
_NW = _NC * _NS
_HALF = _H // 2
_RC = 24
_NCH = _HALF // _RC
_VPR = _W // 16
_EPS2 = 1e-16


def _sc_body(o_hbm, t_hbm, acc_out, cnt_out, bufs, stage, sem0, sem1):
    cid = lax.axis_index("c")
    sid = lax.axis_index("s")
    wid = sid * _NC + cid
    b = wid // 2
    half = wid % 2
    sems = (sem0, sem1)

    def issue(slot, ci):
        r0 = half * _HALF + ci * _RC
        hs = []
        for a, arr in enumerate((o_hbm, t_hbm)):
            for c in range(_C):
                hs.append(pltpu.async_copy(
                    arr.at[b, c, pl.ds(r0, _RC), :],
                    bufs.at[slot, a * _C + c], sems[slot]))
        return hs

    def compute(slot, carry):
        def row(i, rcarry):
            def vec(j, vcarry):
                acc, cnt = vcarry
                sl = pl.ds(j * 16, 16)
                o0 = bufs[slot, 0, i, sl]
                o1 = bufs[slot, 1, i, sl]
                o2 = bufs[slot, 2, i, sl]
                t0 = bufs[slot, 3, i, sl]
                t1 = bufs[slot, 4, i, sl]
                t2 = bufs[slot, 5, i, sl]
                dot = o0 * t0 + o1 * t1 + o2 * t2
                no2 = o0 * o0 + o1 * o1 + o2 * o2
                nt2 = t0 * t0 + t1 * t1 + t2 * t2
                p = jnp.maximum(no2, _EPS2) * jnp.maximum(nt2, _EPS2)
                iv = lax.bitcast_convert_type(p, jnp.int32)
                iv = 0x5F3759DF - lax.shift_right_arithmetic(iv, 1)
                y = lax.bitcast_convert_type(iv, jnp.float32)
                ph = 0.5 * p
                y = y * (1.5 - ph * y * y)
                y = y * (1.5 - ph * y * y)
                y = y * (1.5 - ph * y * y)
                m = nt2 > 0.0
                acc = acc + jnp.where(m, 1.0 - dot * y, 0.0)
                cnt = cnt + jnp.where(m, 1.0, 0.0)
                return acc, cnt
            return lax.fori_loop(0, _VPR, vec, rcarry)
        return lax.fori_loop(0, _RC, row, carry)

    zero = jnp.zeros((16,), jnp.float32)
    carry = (zero, zero)
    pending = issue(0, 0)
    for ci in range(_NCH):
        slot = ci % 2
        for hh in pending:
            hh.wait()
        if ci + 1 < _NCH:
            pending = issue((ci + 1) % 2, ci + 1)
        carry = compute(slot, carry)
    acc, cnt = carry
    stage[...] = acc
    pltpu.sync_copy(stage, acc_out.at[wid])
    stage[...] = cnt
    pltpu.sync_copy(stage, cnt_out.at[wid])


def _fin_body(a_ref, c_ref, out_ref):
    loss = jnp.sum(a_ref[...]) / jnp.sum(c_ref[...])
    out_ref[...] = loss.reshape(1, 1)


def kernel(output, target):
    mesh = plsc.VectorSubcoreMesh(core_axis_name="c", subcore_axis_name="s")
    sc = functools.partial(
        pl.kernel,
        mesh=mesh,
        out_type=[
            jax.ShapeDtypeStruct((_NW, 16), jnp.float32),
            jax.ShapeDtypeStruct((_NW, 16), jnp.float32),
        ],
        scratch_types=[
            pltpu.VMEM((2, 6, _RC, _W), jnp.float32),
            pltpu.VMEM((16,), jnp.float32),
            pltpu.SemaphoreType.DMA,
            pltpu.SemaphoreType.DMA,
        ],
    )(_sc_body)
    acc_p, cnt_p = sc(output, target)
    out = pl.pallas_call(
        _fin_body,
        out_shape=jax.ShapeDtypeStruct((1, 1), jnp.float32),
    )(acc_p, cnt_p)
    return out[0, 0]
